# baseline (speedup 1.0000x reference)
import jax
import jax.numpy as jnp
from jax.experimental import pallas as pl
from jax.experimental.pallas import tpu as pltpu
from jax.experimental.pallas import tpu_sc as plsc

N = 20000
K = 2000
KP = 2048
B = 256
NB = KP // B
MAX_DET = 300
DET = 512
G = 100
GP = 128
CONF_THRES = 0.25
IOU_THRES = 0.45

TD = 16
NC = 2
NS = 16
NW = NC * NS
BPW = KP // NW


def _sc_gather_body(table_hbm, idx_hbm, out_hbm, idx_v, rows_v, sem):
    wid = jax.lax.axis_index("s") * NC + jax.lax.axis_index("c")
    base = wid * BPW
    pltpu.sync_copy(idx_hbm.at[pl.ds(base, BPW)], idx_v)
    pltpu.async_copy(table_hbm.at[idx_v], rows_v, sem).wait()
    pltpu.sync_copy(rows_v, out_hbm.at[pl.ds(base, BPW)])


def _nms_body(col_ref, row_ref, gt_ref, det_ref, gtb_ref,
              keep_ref, keepc_ref, posc_ref, mask_ref):
    f32 = jnp.float32
    lane_all = jax.lax.broadcasted_iota(jnp.int32, (1, KP), 1)
    jloc = jax.lax.broadcasted_iota(jnp.int32, (1, B), 1)

    x1r = row_ref[0:1, :]
    y1r = row_ref[1:2, :]
    x2r = row_ref[2:3, :]
    y2r = row_ref[3:4, :]
    scr = row_ref[4:5, :]
    arear = row_ref[5:6, :]

    keep_ref[0:1, :] = jnp.where((scr > 0.0) & (lane_all < K), 1.0, 0.0)

    cnt = jnp.zeros((1, 1), f32)
    for t in range(NB):
        s = t * B
        x1c = col_ref[s:s + B, 0:1]
        y1c = col_ref[s:s + B, 1:2]
        x2c = col_ref[s:s + B, 2:3]
        y2c = col_ref[s:s + B, 3:4]
        areac = col_ref[s:s + B, 5:6]
        w = jnp.maximum(jnp.minimum(x2c, x2r) - jnp.maximum(x1c, x1r), 0.0)
        h = jnp.maximum(jnp.minimum(y2c, y2r) - jnp.maximum(y1c, y1r), 0.0)
        inter = w * h
        iou = inter / jnp.maximum(areac + arear - inter, 1e-9)
        mask_ref[...] = jnp.where(iou > IOU_THRES, 1.0, 0.0)

        def inner(i, carry):
            kslice, cnt_in = carry
            kg = jnp.sum(kslice * jnp.where(jloc == i, 1.0, 0.0),
                         keepdims=True)
            mrow = mask_ref[pl.ds(i, 1), s:s + B]
            sup = mrow * jnp.where(jloc > i, 1.0, 0.0) * kg
            keepc_ref[pl.ds(s + i, 1), 0:1] = kg
            posc_ref[pl.ds(s + i, 1), 0:1] = cnt_in
            return kslice * (1.0 - sup), cnt_in + kg

        kslice, cnt = jax.lax.fori_loop(
            0, B, inner, (keep_ref[0:1, s:s + B], cnt))
        keep_ref[0:1, s:s + B] = kslice

        kc = keepc_ref[s:s + B, 0:1]
        supall = jnp.max(mask_ref[...] * kc, axis=0, keepdims=True)
        laterf = jnp.where(lane_all >= s + B, 1.0, 0.0)
        keep_ref[0:1, :] = keep_ref[0:1, :] * (1.0 - supall * laterf)

    nk = cnt
    srow = jax.lax.broadcasted_iota(jnp.int32, (1, DET), 1)
    srow_f = srow.astype(f32)
    pc = posc_ref[...]
    kcf = keepc_ref[...]
    onehot = jnp.where(
        (pc == srow_f) & (kcf > 0.0) & (srow < MAX_DET), 1.0, 0.0)
    x1d = jnp.sum(onehot * col_ref[:, 0:1], axis=0, keepdims=True)
    y1d = jnp.sum(onehot * col_ref[:, 1:2], axis=0, keepdims=True)
    x2d = jnp.sum(onehot * col_ref[:, 2:3], axis=0, keepdims=True)
    y2d = jnp.sum(onehot * col_ref[:, 3:4], axis=0, keepdims=True)
    sd = jnp.sum(onehot * col_ref[:, 4:5], axis=0, keepdims=True)
    vr = jnp.where(srow_f < jnp.minimum(nk, float(MAX_DET)), 1.0, 0.0)
    det_ref[0:1, :] = x1d
    det_ref[1:2, :] = y1d
    det_ref[2:3, :] = x2d
    det_ref[3:4, :] = y2d
    det_ref[4:5, :] = sd
    det_ref[5:6, :] = vr
    det_ref[6:7, :] = jnp.zeros((1, DET), f32)
    det_ref[7:8, :] = jnp.zeros((1, DET), f32)

    aread = (x2d - x1d) * (y2d - y1d)
    gx1 = gt_ref[:, 0:1]
    gy1 = gt_ref[:, 1:2]
    gx2 = gt_ref[:, 2:3]
    gy2 = gt_ref[:, 3:4]
    ga = gt_ref[:, 4:5]
    wg = jnp.maximum(jnp.minimum(gx2, x2d) - jnp.maximum(gx1, x1d), 0.0)
    hg = jnp.maximum(jnp.minimum(gy2, y2d) - jnp.maximum(gy1, y1d), 0.0)
    ig = wg * hg
    ioug = ig / jnp.maximum(ga + aread - ig, 1e-9)
    ioug = jnp.where(vr > 0.0, ioug, 0.0)
    best = jnp.max(ioug, axis=1, keepdims=True)
    gtb_ref[...] = jnp.broadcast_to(best, (GP, 8))


def kernel(boxes, scores, gt_boxes):
    f32 = jnp.float32
    xy = boxes[:, :2] * 512.0
    wh = boxes[:, 2:] * 64.0 + 1.0
    box_xyxy = jnp.concatenate([xy, xy + wh], axis=-1)
    masked = jnp.where(scores > CONF_THRES, scores, -1.0)
    top_scores, top_idx = jax.lax.top_k(masked, K)

    area_all = (box_xyxy[:, 2] - box_xyxy[:, 0]) * (box_xyxy[:, 3] - box_xyxy[:, 1])
    table = jnp.zeros((N, TD), f32)
    table = table.at[:, :4].set(box_xyxy)
    table = table.at[:, 4].set(masked)
    table = table.at[:, 5].set(area_all)

    idx = jnp.zeros((KP,), jnp.int32).at[:K].set(top_idx.astype(jnp.int32))

    gathered = pl.kernel(
        _sc_gather_body,
        out_type=jax.ShapeDtypeStruct((KP, TD), f32),
        scratch_types=[
            pltpu.VMEM((BPW,), jnp.int32),
            pltpu.VMEM((BPW, TD), f32),
            pltpu.SemaphoreType.DMA,
        ],
        mesh=plsc.VectorSubcoreMesh(core_axis_name="c", subcore_axis_name="s"),
        compiler_params=pltpu.CompilerParams(use_tc_tiling_on_sc=False),
    )(table, idx)

    colmat = gathered[:, :8]
    rowmat = colmat.T

    gxy = gt_boxes[:, :2] * 512.0
    gwh = gt_boxes[:, 2:] * 64.0 + 1.0
    gt_xyxy = jnp.concatenate([gxy, gxy + gwh], axis=-1)
    garea = (gt_xyxy[:, 2] - gt_xyxy[:, 0]) * (gt_xyxy[:, 3] - gt_xyxy[:, 1])
    gtmat = jnp.zeros((GP, 8), f32)
    gtmat = gtmat.at[:G, :4].set(gt_xyxy)
    gtmat = gtmat.at[:G, 4].set(garea)

    det, gtb = pl.pallas_call(
        _nms_body,
        out_shape=[
            jax.ShapeDtypeStruct((8, DET), f32),
            jax.ShapeDtypeStruct((GP, 8), f32),
        ],
        scratch_shapes=[
            pltpu.VMEM((1, KP), f32),
            pltpu.VMEM((KP, 1), f32),
            pltpu.VMEM((KP, 1), f32),
            pltpu.VMEM((B, KP), f32),
        ],
    )(colmat, rowmat, gtmat)

    det_boxes = det[:4, :MAX_DET].T
    det_scores = det[4, :MAX_DET]
    best_gt_iou = gtb[:G, 0]
    return jnp.concatenate([det_boxes.reshape(-1), det_scores, best_gt_iou])


# speedup vs baseline: 2.4444x; 2.4444x over previous
import jax
import jax.numpy as jnp
from jax.experimental import pallas as pl
from jax.experimental.pallas import tpu as pltpu
from jax.experimental.pallas import tpu_sc as plsc

N = 20000
K = 2000
KP = 2048
B = 256
NB = KP // B
MAX_DET = 300
DET = 512
G = 100
GP = 128
CONF_THRES = 0.25
IOU_THRES = 0.45

TD = 16
NC = 2
NS = 16
NW = NC * NS
BPW = KP // NW


def _sc_gather_body(table_hbm, idx_hbm, out_hbm, idx_v, rows_v, sem):
    wid = jax.lax.axis_index("s") * NC + jax.lax.axis_index("c")
    base = wid * BPW
    pltpu.sync_copy(idx_hbm.at[pl.ds(base, BPW)], idx_v)
    pltpu.async_copy(table_hbm.at[idx_v], rows_v, sem).wait()
    pltpu.sync_copy(rows_v, out_hbm.at[pl.ds(base, BPW)])


def _nms_body(col_ref, row_ref, gt_ref, det_ref, gtb_ref,
              keep_ref, keepc_ref, posc_ref, mask_ref):
    f32 = jnp.float32
    lane_all = jax.lax.broadcasted_iota(jnp.int32, (1, KP), 1)
    jloc = jax.lax.broadcasted_iota(jnp.int32, (1, B), 1)

    x1r = row_ref[0:1, :]
    y1r = row_ref[1:2, :]
    x2r = row_ref[2:3, :]
    y2r = row_ref[3:4, :]
    scr = row_ref[4:5, :]
    arear = row_ref[5:6, :]

    keep_ref[0:1, :] = jnp.where((scr > 0.0) & (lane_all < K), 1.0, 0.0)

    rowio = jax.lax.broadcasted_iota(jnp.int32, (B, B), 0)
    laneio = jax.lax.broadcasted_iota(jnp.int32, (B, B), 1)
    tri = jnp.where(rowio < laneio, 1.0, 0.0)
    eye = jnp.where(rowio == laneio, 1.0, 0.0)
    trir = jnp.where(laneio < rowio, 1.0, 0.0)

    cnt = jnp.zeros((1, 1), f32)
    for t in range(NB):
        s = t * B
        x1c = col_ref[s:s + B, 0:1]
        y1c = col_ref[s:s + B, 1:2]
        x2c = col_ref[s:s + B, 2:3]
        y2c = col_ref[s:s + B, 3:4]
        areac = col_ref[s:s + B, 5:6]
        w = jnp.maximum(jnp.minimum(x2c, x2r) - jnp.maximum(x1c, x1r), 0.0)
        h = jnp.maximum(jnp.minimum(y2c, y2r) - jnp.maximum(y1c, y1r), 0.0)
        inter = w * h
        iou = inter / jnp.maximum(areac + arear - inter, 1e-9)
        mask_ref[...] = jnp.where(iou > IOU_THRES, 1.0, 0.0)

        am = mask_ref[:, s:s + B] * tri
        pre = keep_ref[0:1, s:s + B]

        def f(srow):
            scol = jnp.sum(srow * eye, axis=1, keepdims=True)
            sup = jnp.max(am * scol, axis=0, keepdims=True)
            return pre * (1.0 - sup)

        lo = f(pre)
        hi = f(lo)

        def conv_cond(c):
            lo_, hi_ = c
            return jnp.sum(hi_ - lo_) > 0.0

        def conv_body(c):
            _, hi_ = c
            lo2 = f(hi_)
            return lo2, f(lo2)

        lo, hi = jax.lax.while_loop(conv_cond, conv_body, (lo, hi))
        kslice = lo
        keep_ref[0:1, s:s + B] = kslice

        kcol = jnp.sum(kslice * eye, axis=1, keepdims=True)
        keepc_ref[s:s + B, 0:1] = kcol
        poscol = jnp.sum(trir * kslice, axis=1, keepdims=True)
        posc_ref[s:s + B, 0:1] = poscol + cnt
        cnt = cnt + jnp.sum(kslice, keepdims=True)

        supall = jnp.max(mask_ref[...] * kcol, axis=0, keepdims=True)
        laterf = jnp.where(lane_all >= s + B, 1.0, 0.0)
        keep_ref[0:1, :] = keep_ref[0:1, :] * (1.0 - supall * laterf)

    nk = cnt
    srow = jax.lax.broadcasted_iota(jnp.int32, (1, DET), 1)
    srow_f = srow.astype(f32)
    pc = posc_ref[...]
    kcf = keepc_ref[...]
    onehot = jnp.where(
        (pc == srow_f) & (kcf > 0.0) & (srow < MAX_DET), 1.0, 0.0)
    x1d = jnp.sum(onehot * col_ref[:, 0:1], axis=0, keepdims=True)
    y1d = jnp.sum(onehot * col_ref[:, 1:2], axis=0, keepdims=True)
    x2d = jnp.sum(onehot * col_ref[:, 2:3], axis=0, keepdims=True)
    y2d = jnp.sum(onehot * col_ref[:, 3:4], axis=0, keepdims=True)
    sd = jnp.sum(onehot * col_ref[:, 4:5], axis=0, keepdims=True)
    vr = jnp.where(srow_f < jnp.minimum(nk, float(MAX_DET)), 1.0, 0.0)
    det_ref[0:1, :] = x1d
    det_ref[1:2, :] = y1d
    det_ref[2:3, :] = x2d
    det_ref[3:4, :] = y2d
    det_ref[4:5, :] = sd
    det_ref[5:6, :] = vr
    det_ref[6:7, :] = jnp.zeros((1, DET), f32)
    det_ref[7:8, :] = jnp.zeros((1, DET), f32)

    aread = (x2d - x1d) * (y2d - y1d)
    gx1 = gt_ref[:, 0:1]
    gy1 = gt_ref[:, 1:2]
    gx2 = gt_ref[:, 2:3]
    gy2 = gt_ref[:, 3:4]
    ga = gt_ref[:, 4:5]
    wg = jnp.maximum(jnp.minimum(gx2, x2d) - jnp.maximum(gx1, x1d), 0.0)
    hg = jnp.maximum(jnp.minimum(gy2, y2d) - jnp.maximum(gy1, y1d), 0.0)
    ig = wg * hg
    ioug = ig / jnp.maximum(ga + aread - ig, 1e-9)
    ioug = jnp.where(vr > 0.0, ioug, 0.0)
    best = jnp.max(ioug, axis=1, keepdims=True)
    gtb_ref[...] = jnp.broadcast_to(best, (GP, 8))


def kernel(boxes, scores, gt_boxes):
    f32 = jnp.float32
    xy = boxes[:, :2] * 512.0
    wh = boxes[:, 2:] * 64.0 + 1.0
    box_xyxy = jnp.concatenate([xy, xy + wh], axis=-1)
    masked = jnp.where(scores > CONF_THRES, scores, -1.0)
    top_scores, top_idx = jax.lax.top_k(masked, K)

    area_all = (box_xyxy[:, 2] - box_xyxy[:, 0]) * (box_xyxy[:, 3] - box_xyxy[:, 1])
    table = jnp.zeros((N, TD), f32)
    table = table.at[:, :4].set(box_xyxy)
    table = table.at[:, 4].set(masked)
    table = table.at[:, 5].set(area_all)

    idx = jnp.zeros((KP,), jnp.int32).at[:K].set(top_idx.astype(jnp.int32))

    gathered = pl.kernel(
        _sc_gather_body,
        out_type=jax.ShapeDtypeStruct((KP, TD), f32),
        scratch_types=[
            pltpu.VMEM((BPW,), jnp.int32),
            pltpu.VMEM((BPW, TD), f32),
            pltpu.SemaphoreType.DMA,
        ],
        mesh=plsc.VectorSubcoreMesh(core_axis_name="c", subcore_axis_name="s"),
        compiler_params=pltpu.CompilerParams(use_tc_tiling_on_sc=False),
    )(table, idx)

    colmat = gathered[:, :8]
    rowmat = colmat.T

    gxy = gt_boxes[:, :2] * 512.0
    gwh = gt_boxes[:, 2:] * 64.0 + 1.0
    gt_xyxy = jnp.concatenate([gxy, gxy + gwh], axis=-1)
    garea = (gt_xyxy[:, 2] - gt_xyxy[:, 0]) * (gt_xyxy[:, 3] - gt_xyxy[:, 1])
    gtmat = jnp.zeros((GP, 8), f32)
    gtmat = gtmat.at[:G, :4].set(gt_xyxy)
    gtmat = gtmat.at[:G, 4].set(garea)

    det, gtb = pl.pallas_call(
        _nms_body,
        out_shape=[
            jax.ShapeDtypeStruct((8, DET), f32),
            jax.ShapeDtypeStruct((GP, 8), f32),
        ],
        scratch_shapes=[
            pltpu.VMEM((1, KP), f32),
            pltpu.VMEM((KP, 1), f32),
            pltpu.VMEM((KP, 1), f32),
            pltpu.VMEM((B, KP), f32),
        ],
    )(colmat, rowmat, gtmat)

    det_boxes = det[:4, :MAX_DET].T
    det_scores = det[4, :MAX_DET]
    best_gt_iou = gtb[:G, 0]
    return jnp.concatenate([det_boxes.reshape(-1), det_scores, best_gt_iou])
